# Initial kernel scaffold; baseline (speedup 1.0000x reference)
import jax
import jax.numpy as jnp
from jax.experimental import pallas as pl


def kernel(node_states, adjacency_list_0, adjacency_list_1, node_to_graph_idx, W_self, W0, W1, b):
    raise NotImplementedError("write your pallas kernel here")


# speedup vs baseline: 6.2694x; 6.2694x over previous
import functools

import jax
import jax.numpy as jnp
from jax import lax
from jax.experimental import pallas as pl
from jax.experimental.pallas import tpu as pltpu
from jax.experimental.pallas import tpu_sc as plsc

N = 10000
D = 128
E = 160000
NUM_CORES = 2
NUM_TILES = 16
EDGES_PER_TILE = E // NUM_TILES
CHUNK = 80
NUM_CHUNKS = EDGES_PER_TILE // CHUNK
N_PAD = 10240
ROWS_PER_TILE = N_PAD // NUM_TILES


def _sc_aggregate(node_states, src_all, dst_all, zeros):
    mesh = plsc.VectorSubcoreMesh(core_axis_name="c", subcore_axis_name="s")

    @functools.partial(
        pl.kernel,
        mesh=mesh,
        out_type=jax.ShapeDtypeStruct((NUM_CORES * N_PAD, D), jnp.float32),
        scratch_types=[
            pltpu.VMEM_SHARED((N_PAD, D), jnp.float32),
            pltpu.VMEM((NUM_CHUNKS, CHUNK), jnp.int32),
            pltpu.VMEM((NUM_CHUNKS, CHUNK), jnp.int32),
            pltpu.VMEM((CHUNK, D), jnp.float32),
            pltpu.SemaphoreType.DMA,
        ],
    )
    def agg_kernel(x_hbm, src_hbm, dst_hbm, zeros_hbm, out_hbm,
                   s_sh, src_idx, dst_idx, rows, sem):
        cid = lax.axis_index("c")
        sid = lax.axis_index("s")
        wid = cid * NUM_TILES + sid
        r0 = sid * ROWS_PER_TILE

        pltpu.sync_copy(zeros_hbm.at[pl.ds(r0, ROWS_PER_TILE)],
                        s_sh.at[pl.ds(r0, ROWS_PER_TILE)])
        pltpu.sync_copy(src_hbm.at[wid], src_idx)
        pltpu.sync_copy(dst_hbm.at[wid], dst_idx)
        plsc.subcore_barrier()

        def body(j, carry):
            pltpu.async_copy(x_hbm.at[src_idx.at[j]], rows, sem).wait()
            pltpu.sync_copy(rows, s_sh.at[dst_idx.at[j]], add=True)
            return carry

        lax.fori_loop(0, NUM_CHUNKS, body, 0)
        plsc.subcore_barrier()
        pltpu.sync_copy(s_sh.at[pl.ds(r0, ROWS_PER_TILE)],
                        out_hbm.at[pl.ds(cid * N_PAD + r0, ROWS_PER_TILE)])

    return agg_kernel(node_states, src_all, dst_all, zeros)


BLOCK_M = 1000


def _tc_combine(x, s0, s1, w_self, w0, w1, b2d):
    def body(x_ref, s0_ref, s1_ref, ws_ref, w0_ref, w1_ref, b_ref, o_ref):
        acc = jnp.dot(x_ref[...], ws_ref[...], preferred_element_type=jnp.float32)
        acc = acc + jnp.dot(s0_ref[...], w0_ref[...], preferred_element_type=jnp.float32)
        acc = acc + jnp.dot(s1_ref[...], w1_ref[...], preferred_element_type=jnp.float32)
        o_ref[...] = jnp.maximum(acc + b_ref[...], 0.0)

    return pl.pallas_call(
        body,
        grid=(N // BLOCK_M,),
        in_specs=[
            pl.BlockSpec((BLOCK_M, D), lambda i: (i, 0)),
            pl.BlockSpec((BLOCK_M, D), lambda i: (i, 0)),
            pl.BlockSpec((BLOCK_M, D), lambda i: (i, 0)),
            pl.BlockSpec((D, D), lambda i: (0, 0)),
            pl.BlockSpec((D, D), lambda i: (0, 0)),
            pl.BlockSpec((D, D), lambda i: (0, 0)),
            pl.BlockSpec((1, D), lambda i: (0, 0)),
        ],
        out_specs=pl.BlockSpec((BLOCK_M, D), lambda i: (i, 0)),
        out_shape=jax.ShapeDtypeStruct((N, D), jnp.float32),
    )(x, s0, s1, w_self, w0, w1, b2d)


def kernel(node_states, adjacency_list_0, adjacency_list_1, node_to_graph_idx,
           W_self, W0, W1, b):
    src_all = jnp.concatenate(
        [adjacency_list_0[:, 0], adjacency_list_1[:, 0]]
    ).reshape(NUM_CORES * NUM_TILES, NUM_CHUNKS, CHUNK)
    dst_all = jnp.concatenate(
        [adjacency_list_0[:, 1], adjacency_list_1[:, 1]]
    ).reshape(NUM_CORES * NUM_TILES, NUM_CHUNKS, CHUNK)
    zeros = jnp.zeros((N_PAD, D), jnp.float32)
    aggs = _sc_aggregate(node_states, src_all, dst_all, zeros)
    return _tc_combine(node_states, aggs[:N], aggs[N_PAD:N_PAD + N], W_self, W0, W1,
                       b.reshape(1, D))


# speedup vs baseline: 9.8086x; 1.5645x over previous
import functools

import jax
import jax.numpy as jnp
from jax import lax
from jax.experimental import pallas as pl
from jax.experimental.pallas import tpu as pltpu
from jax.experimental.pallas import tpu_sc as plsc

N = 10000
D = 128
E = 160000
NUM_CORES = 2
NUM_TILES = 16
NUM_WORKERS = NUM_CORES * NUM_TILES
CHUNK = 100
NUM_CHUNKS = 50
EDGES_PER_TILE = CHUNK * NUM_CHUNKS
N_ACC = 10112
ROWS_PER_TILE = N_ACC // NUM_TILES


def _sc_aggregate(node_states, src_all, dst_all, zeros):
    mesh = plsc.VectorSubcoreMesh(core_axis_name="c", subcore_axis_name="s")

    @functools.partial(
        pl.kernel,
        mesh=mesh,
        out_type=jax.ShapeDtypeStruct((2 * N_ACC, D), jnp.float32),
        scratch_types=[
            pltpu.VMEM_SHARED((N_ACC, D), jnp.float32),
            pltpu.VMEM((NUM_CHUNKS, CHUNK), jnp.int32),
            pltpu.VMEM((NUM_CHUNKS, CHUNK), jnp.int32),
            pltpu.VMEM((CHUNK, D), jnp.float32),
            pltpu.VMEM((CHUNK, D), jnp.float32),
            pltpu.SemaphoreType.DMA,
            pltpu.SemaphoreType.DMA,
        ],
    )
    def agg_kernel(x_hbm, src_hbm, dst_hbm, zeros_hbm, out_hbm,
                   acc, src_idx, dst_idx, rows0, rows1, sem0, sem1):
        cid = lax.axis_index("c")
        sid = lax.axis_index("s")
        wid = cid * NUM_TILES + sid
        r0 = sid * ROWS_PER_TILE

        bufs = (rows0, rows1)
        sems = (sem0, sem1)

        def start(j, b):
            pltpu.async_copy(x_hbm.at[src_idx.at[j]], bufs[b], sems[b])

        def finish(j, b):
            pltpu.make_async_copy(x_hbm.at[src_idx.at[j]], bufs[b],
                                  sems[b]).wait()
            pltpu.sync_copy(bufs[b], acc.at[dst_idx.at[j]], add=True)

        pltpu.sync_copy(zeros_hbm.at[pl.ds(r0, ROWS_PER_TILE)],
                        acc.at[pl.ds(r0, ROWS_PER_TILE)])
        plsc.subcore_barrier()

        for p in range(2):
            pltpu.sync_copy(src_hbm.at[2 * wid + p], src_idx)
            pltpu.sync_copy(dst_hbm.at[2 * wid + p], dst_idx)

            start(0, 0)
            start(1, 1)

            def body(i, carry):
                j = 2 * i
                finish(j, 0)
                start(j + 2, 0)
                finish(j + 1, 1)
                start(j + 3, 1)
                return carry

            lax.fori_loop(0, NUM_CHUNKS // 2 - 1, body, 0)
            finish(NUM_CHUNKS - 2, 0)
            finish(NUM_CHUNKS - 1, 1)

        plsc.subcore_barrier()
        pltpu.sync_copy(acc.at[pl.ds(r0, ROWS_PER_TILE)],
                        out_hbm.at[pl.ds(cid * N_ACC + r0, ROWS_PER_TILE)])

    return agg_kernel(node_states, src_all, dst_all, zeros)


BLOCK_M = 1000


def _tc_combine(x, s0, s1, w_self, w0, w1, b2d):
    def body(x_ref, s0_ref, s1_ref, ws_ref, w0_ref, w1_ref, b_ref, o_ref):
        acc = jnp.dot(x_ref[...], ws_ref[...], preferred_element_type=jnp.float32)
        acc = acc + jnp.dot(s0_ref[...], w0_ref[...], preferred_element_type=jnp.float32)
        acc = acc + jnp.dot(s1_ref[...], w1_ref[...], preferred_element_type=jnp.float32)
        o_ref[...] = jnp.maximum(acc + b_ref[...], 0.0)

    return pl.pallas_call(
        body,
        grid=(N // BLOCK_M,),
        in_specs=[
            pl.BlockSpec((BLOCK_M, D), lambda i: (i, 0)),
            pl.BlockSpec((BLOCK_M, D), lambda i: (i, 0)),
            pl.BlockSpec((BLOCK_M, D), lambda i: (i, 0)),
            pl.BlockSpec((D, D), lambda i: (0, 0)),
            pl.BlockSpec((D, D), lambda i: (0, 0)),
            pl.BlockSpec((D, D), lambda i: (0, 0)),
            pl.BlockSpec((1, D), lambda i: (0, 0)),
        ],
        out_specs=pl.BlockSpec((BLOCK_M, D), lambda i: (i, 0)),
        out_shape=jax.ShapeDtypeStruct((N, D), jnp.float32),
    )(x, s0, s1, w_self, w0, w1, b2d)


def kernel(node_states, adjacency_list_0, adjacency_list_1, node_to_graph_idx,
           W_self, W0, W1, b):
    src_all = jnp.concatenate(
        [adjacency_list_0[:, 0], adjacency_list_1[:, 0]]
    ).reshape(2 * NUM_WORKERS, NUM_CHUNKS, CHUNK)
    dst_all = jnp.concatenate(
        [adjacency_list_0[:, 1], adjacency_list_1[:, 1]]
    ).reshape(2 * NUM_WORKERS, NUM_CHUNKS, CHUNK)
    zeros = jnp.zeros((N_ACC, D), jnp.float32)
    aggs = _sc_aggregate(node_states, src_all, dst_all, zeros)
    return _tc_combine(node_states, aggs[:N], aggs[N_ACC:N_ACC + N],
                       W_self, W0, W1, b.reshape(1, D))


# speedup vs baseline: 9.9647x; 1.0159x over previous
import functools

import jax
import jax.numpy as jnp
from jax import lax
from jax.experimental import pallas as pl
from jax.experimental.pallas import tpu as pltpu
from jax.experimental.pallas import tpu_sc as plsc

N = 10000
D = 128
E = 160000
NUM_CORES = 2
NUM_TILES = 16
NUM_WORKERS = NUM_CORES * NUM_TILES
CHUNK = 125
NUM_CHUNKS = 40
EDGES_PER_TILE = CHUNK * NUM_CHUNKS
N_ACC = 10112
ROWS_PER_TILE = N_ACC // NUM_TILES


def _sc_aggregate(node_states, src_all, dst_all, zeros):
    mesh = plsc.VectorSubcoreMesh(core_axis_name="c", subcore_axis_name="s")

    @functools.partial(
        pl.kernel,
        mesh=mesh,
        out_type=jax.ShapeDtypeStruct((2 * N_ACC, D), jnp.float32),
        scratch_types=[
            pltpu.VMEM_SHARED((N_ACC, D), jnp.float32),
            pltpu.VMEM((NUM_CHUNKS, CHUNK), jnp.int32),
            pltpu.VMEM((NUM_CHUNKS, CHUNK), jnp.int32),
            pltpu.VMEM((CHUNK, D), jnp.float32),
            pltpu.VMEM((CHUNK, D), jnp.float32),
            pltpu.SemaphoreType.DMA,
            pltpu.SemaphoreType.DMA,
        ],
    )
    def agg_kernel(x_hbm, src_hbm, dst_hbm, zeros_hbm, out_hbm,
                   acc, src_idx, dst_idx, rows0, rows1, sem0, sem1):
        cid = lax.axis_index("c")
        sid = lax.axis_index("s")
        wid = cid * NUM_TILES + sid
        r0 = sid * ROWS_PER_TILE

        bufs = (rows0, rows1)
        sems = (sem0, sem1)

        def start(j, b):
            pltpu.async_copy(x_hbm.at[src_idx.at[j]], bufs[b], sems[b])

        def finish(j, b):
            pltpu.make_async_copy(x_hbm.at[src_idx.at[j]], bufs[b],
                                  sems[b]).wait()
            pltpu.sync_copy(bufs[b], acc.at[dst_idx.at[j]], add=True)

        pltpu.sync_copy(zeros_hbm.at[pl.ds(r0, ROWS_PER_TILE)],
                        acc.at[pl.ds(r0, ROWS_PER_TILE)])
        plsc.subcore_barrier()

        for p in range(2):
            pltpu.sync_copy(src_hbm.at[2 * wid + p], src_idx)
            pltpu.sync_copy(dst_hbm.at[2 * wid + p], dst_idx)

            start(0, 0)
            start(1, 1)

            def body(i, carry):
                j = 2 * i
                finish(j, 0)
                start(j + 2, 0)
                finish(j + 1, 1)
                start(j + 3, 1)
                return carry

            lax.fori_loop(0, NUM_CHUNKS // 2 - 1, body, 0)
            finish(NUM_CHUNKS - 2, 0)
            finish(NUM_CHUNKS - 1, 1)

        plsc.subcore_barrier()
        pltpu.sync_copy(acc.at[pl.ds(r0, ROWS_PER_TILE)],
                        out_hbm.at[pl.ds(cid * N_ACC + r0, ROWS_PER_TILE)])

    return agg_kernel(node_states, src_all, dst_all, zeros)


BLOCK_M = 1000


def _tc_combine(x, s0, s1, w_self, w0, w1, b2d):
    def body(x_ref, s0_ref, s1_ref, ws_ref, w0_ref, w1_ref, b_ref, o_ref):
        acc = jnp.dot(x_ref[...], ws_ref[...], preferred_element_type=jnp.float32)
        acc = acc + jnp.dot(s0_ref[...], w0_ref[...], preferred_element_type=jnp.float32)
        acc = acc + jnp.dot(s1_ref[...], w1_ref[...], preferred_element_type=jnp.float32)
        o_ref[...] = jnp.maximum(acc + b_ref[...], 0.0)

    return pl.pallas_call(
        body,
        grid=(N // BLOCK_M,),
        in_specs=[
            pl.BlockSpec((BLOCK_M, D), lambda i: (i, 0)),
            pl.BlockSpec((BLOCK_M, D), lambda i: (i, 0)),
            pl.BlockSpec((BLOCK_M, D), lambda i: (i, 0)),
            pl.BlockSpec((D, D), lambda i: (0, 0)),
            pl.BlockSpec((D, D), lambda i: (0, 0)),
            pl.BlockSpec((D, D), lambda i: (0, 0)),
            pl.BlockSpec((1, D), lambda i: (0, 0)),
        ],
        out_specs=pl.BlockSpec((BLOCK_M, D), lambda i: (i, 0)),
        out_shape=jax.ShapeDtypeStruct((N, D), jnp.float32),
    )(x, s0, s1, w_self, w0, w1, b2d)


def kernel(node_states, adjacency_list_0, adjacency_list_1, node_to_graph_idx,
           W_self, W0, W1, b):
    src_all = jnp.concatenate(
        [adjacency_list_0[:, 0], adjacency_list_1[:, 0]]
    ).reshape(2 * NUM_WORKERS, NUM_CHUNKS, CHUNK)
    dst_all = jnp.concatenate(
        [adjacency_list_0[:, 1], adjacency_list_1[:, 1]]
    ).reshape(2 * NUM_WORKERS, NUM_CHUNKS, CHUNK)
    zeros = jnp.zeros((N_ACC, D), jnp.float32)
    aggs = _sc_aggregate(node_states, src_all, dst_all, zeros)
    return _tc_combine(node_states, aggs[:N], aggs[N_ACC:N_ACC + N],
                       W_self, W0, W1, b.reshape(1, D))
